# baseline (speedup 1.0000x reference)
import jax
import jax.numpy as jnp
from jax import lax
from jax.experimental import pallas as pl
from jax.experimental.pallas import tpu as pltpu
from jax.experimental.pallas import tpu_sc as plsc

N_NODES = 10000
N_PAD = 10240
E_PAD = 163840
DH = 512
NSLICE = 4
WSL = 128
EB = 128
ROWS_T = N_PAD // 16

_MESH = plsc.VectorSubcoreMesh(
    core_axis_name="c", subcore_axis_name="s", num_cores=2, num_subcores=16)


def _deg_body(col_hbm, ew_hbm, out_hbm, colv, ewv, zbuf, acc_sh):
    c = lax.axis_index("c")
    t = lax.axis_index("s")
    zero16 = jnp.zeros((16,), jnp.float32)
    for i in range(ROWS_T // 16):
        zbuf[pl.ds(i * 16, 16)] = zero16
    pltpu.sync_copy(zbuf, acc_sh.at[pl.ds(t * ROWS_T, ROWS_T)])
    npb = (E_PAD // 2) // 16 // EB
    blk0 = (c * 16 + t) * npb
    pltpu.sync_copy(col_hbm.at[pl.ds(blk0, npb)], colv)
    pltpu.sync_copy(ew_hbm.at[pl.ds(blk0, npb)], ewv)
    plsc.subcore_barrier()

    def blk(b, carry):
        pltpu.sync_copy(ewv.at[b], acc_sh.at[colv.at[b]], add=True)
        return carry

    lax.fori_loop(0, npb, blk, 0)
    plsc.subcore_barrier()
    pltpu.sync_copy(acc_sh.at[pl.ds(t * ROWS_T, ROWS_T)],
                    out_hbm.at[pl.ds(c * N_PAD + t * ROWS_T, ROWS_T)])


_deg_call = pl.kernel(
    _deg_body,
    out_type=jax.ShapeDtypeStruct((2 * N_PAD,), jnp.float32),
    mesh=_MESH,
    scratch_types=[
        pltpu.VMEM(((E_PAD // 2) // 16 // EB, EB), jnp.int32),
        pltpu.VMEM(((E_PAD // 2) // 16 // EB, EB), jnp.float32),
        pltpu.VMEM((ROWS_T,), jnp.float32),
        pltpu.VMEM_SHARED((N_PAD,), jnp.float32),
    ],
)


_NBLK = (E_PAD // 16) // EB
_CH = 16


def _agg_body(g_hbm, rows4_hbm, col_hbm, ew_hbm, out_hbm,
              idxv, colv, ewv, buf0, buf1, acc_sh, gs0, gs1):
    c = lax.axis_index("c")
    t = lax.axis_index("s")

    def scale(b, buf):
        def grp(j, cc):
            w16 = ewv[b, pl.ds(j * 16, 16)]
            for l in range(16):
                w = w16[l]
                e = j * 16 + l
                for q in range(8):
                    sl = pl.ds(q * 16, 16)
                    buf[e, sl] = buf[e, sl] * w
            return cc
        lax.fori_loop(0, 8, grp, 0)

    for k in range(2):
        s = 2 * c + k
        srow = s * N_PAD
        pltpu.sync_copy(g_hbm.at[pl.ds(srow + t * ROWS_T, ROWS_T)],
                        acc_sh.at[pl.ds(t * ROWS_T, ROWS_T)])
        plsc.subcore_barrier()

        def chunk(ch, carry0):
            blk0 = t * _NBLK + ch * _CH
            pltpu.sync_copy(
                rows4_hbm.at[pl.ds(s * (E_PAD // EB) + blk0, _CH)], idxv)
            pltpu.sync_copy(col_hbm.at[pl.ds(blk0, _CH)], colv)
            pltpu.sync_copy(ew_hbm.at[pl.ds(blk0, _CH)], ewv)

            pltpu.async_copy(g_hbm.at[idxv.at[0]], buf0, gs0)

            def pair(m, carry):
                b0 = 2 * m
                b1 = 2 * m + 1
                pltpu.async_copy(g_hbm.at[idxv.at[b1]], buf1, gs1)
                pltpu.make_async_copy(g_hbm.at[idxv.at[b0]], buf0, gs0).wait()
                scale(b0, buf0)
                pltpu.sync_copy(buf0, acc_sh.at[colv.at[b0]], add=True)

                @pl.when(b1 + 1 < _CH)
                def _():
                    pltpu.async_copy(g_hbm.at[idxv.at[b1 + 1]], buf0, gs0)

                pltpu.make_async_copy(g_hbm.at[idxv.at[b1]], buf1, gs1).wait()
                scale(b1, buf1)
                pltpu.sync_copy(buf1, acc_sh.at[colv.at[b1]], add=True)
                return carry

            lax.fori_loop(0, _CH // 2, pair, 0)
            return carry0

        lax.fori_loop(0, _NBLK // _CH, chunk, 0)
        plsc.subcore_barrier()
        pltpu.sync_copy(acc_sh.at[pl.ds(t * ROWS_T, ROWS_T)],
                        out_hbm.at[pl.ds(srow + t * ROWS_T, ROWS_T)])
        plsc.subcore_barrier()


_agg_call = pl.kernel(
    _agg_body,
    out_type=jax.ShapeDtypeStruct((NSLICE * N_PAD, WSL), jnp.float32),
    mesh=_MESH,
    scratch_types=[
        pltpu.VMEM((_CH, EB), jnp.int32),
        pltpu.VMEM((_CH, EB), jnp.int32),
        pltpu.VMEM((_CH, EB), jnp.float32),
        pltpu.VMEM((EB, WSL), jnp.float32),
        pltpu.VMEM((EB, WSL), jnp.float32),
        pltpu.VMEM_SHARED((N_PAD, WSL), jnp.float32),
        pltpu.SemaphoreType.DMA,
        pltpu.SemaphoreType.DMA,
    ],
)


_BN = 1024


def _dinv_of(deg_ref):
    return lax.rsqrt(deg_ref[0, :] + deg_ref[1, :] + 1.0)


def _elu(h):
    return jnp.where(h > 0, h, jnp.exp(h) - 1.0)


def _mm1_body(deg_ref, x_ref, w_ref, out_ref):
    dinv = _dinv_of(deg_ref)
    h = jnp.dot(x_ref[...], w_ref[...], preferred_element_type=jnp.float32)
    for tt in range(NSLICE):
        out_ref[tt] = dinv[:, None] * h[:, tt * WSL:(tt + 1) * WSL]


def _mid_body(deg_ref, acc_ref, b_ref, w_ref, out_ref):
    dinv = _dinv_of(deg_ref)
    acts = [
        _elu(dinv[:, None] * acc_ref[si] + b_ref[0, si * WSL:(si + 1) * WSL][None, :])
        for si in range(NSLICE)
    ]
    act = jnp.concatenate(acts, axis=1)
    h = jnp.dot(act, w_ref[...], preferred_element_type=jnp.float32)
    for tt in range(NSLICE):
        out_ref[tt] = dinv[:, None] * h[:, tt * WSL:(tt + 1) * WSL]


def _fin_body(deg_ref, acc_ref, b_ref, w_ref, blin_ref, out_ref):
    dinv = _dinv_of(deg_ref)
    acts = [
        _elu(dinv[:, None] * acc_ref[si] + b_ref[0, si * WSL:(si + 1) * WSL][None, :])
        for si in range(NSLICE)
    ]
    act = jnp.concatenate(acts, axis=1)
    out_ref[...] = (
        jnp.dot(act, w_ref[...], preferred_element_type=jnp.float32)
        + blin_ref[0, :][None, :])


_deg_spec = pl.BlockSpec((2, _BN), lambda i: (0, i))
_acc_spec = pl.BlockSpec((NSLICE, _BN, WSL), lambda i: (0, i, 0))


def _mm1(deg2, x_p, W1):
    return pl.pallas_call(
        _mm1_body,
        grid=(N_PAD // _BN,),
        in_specs=[
            _deg_spec,
            pl.BlockSpec((_BN, 256), lambda i: (i, 0)),
            pl.BlockSpec((256, DH), lambda i: (0, 0)),
        ],
        out_specs=_acc_spec,
        out_shape=jax.ShapeDtypeStruct((NSLICE, N_PAD, WSL), jnp.float32),
    )(deg2, x_p, W1)


def _mid(deg2, acc, b_prev, W):
    return pl.pallas_call(
        _mid_body,
        grid=(N_PAD // _BN,),
        in_specs=[
            _deg_spec,
            _acc_spec,
            pl.BlockSpec((1, DH), lambda i: (0, 0)),
            pl.BlockSpec((DH, DH), lambda i: (0, 0)),
        ],
        out_specs=_acc_spec,
        out_shape=jax.ShapeDtypeStruct((NSLICE, N_PAD, WSL), jnp.float32),
    )(deg2, acc, b_prev, W)


def _fin(deg2, acc, b_prev, W_lin, b_lin):
    return pl.pallas_call(
        _fin_body,
        grid=(N_PAD // _BN,),
        in_specs=[
            _deg_spec,
            _acc_spec,
            pl.BlockSpec((1, DH), lambda i: (0, 0)),
            pl.BlockSpec((DH, 256), lambda i: (0, 0)),
            pl.BlockSpec((1, 256), lambda i: (0, 0)),
        ],
        out_specs=pl.BlockSpec((_BN, 256), lambda i: (i, 0)),
        out_shape=jax.ShapeDtypeStruct((N_PAD, 256), jnp.float32),
    )(deg2, acc, b_prev, W_lin, b_lin)


def kernel(x, edge_index, edge_feats, W1, b1, W_hidden, b_hidden, W_lin, b_lin):
    e = edge_index.shape[1]
    pe = E_PAD - e
    row_p = jnp.pad(edge_index[0], (0, pe))
    col_p = jnp.pad(edge_index[1], (0, pe))
    ew_p = jnp.pad(edge_feats, (0, pe))
    offs = (jnp.arange(NSLICE, dtype=jnp.int32) * N_PAD)[:, None]
    rows4 = (row_p[None, :] + offs).reshape(-1, EB)
    col2 = col_p.reshape(-1, EB)
    ew2 = ew_p.reshape(-1, EB)
    x_p = jnp.pad(x, ((0, N_PAD - N_NODES), (0, 0)))

    deg2 = _deg_call(col2, ew2).reshape(2, N_PAD)

    g = _mm1(deg2, x_p, W1)
    acc = _agg_call(g.reshape(NSLICE * N_PAD, WSL), rows4, col2, ew2)
    acc = acc.reshape(NSLICE, N_PAD, WSL)
    b_prev = b1
    for i in range(W_hidden.shape[0]):
        g = _mid(deg2, acc, b_prev.reshape(1, DH), W_hidden[i])
        acc = _agg_call(g.reshape(NSLICE * N_PAD, WSL), rows4, col2, ew2)
        acc = acc.reshape(NSLICE, N_PAD, WSL)
        b_prev = b_hidden[i]
    out = _fin(deg2, acc, b_prev.reshape(1, DH), W_lin, b_lin.reshape(1, -1))
    return out[:N_NODES]


# speedup vs baseline: 1.0323x; 1.0323x over previous
import jax
import jax.numpy as jnp
from jax import lax
from jax.experimental import pallas as pl
from jax.experimental.pallas import tpu as pltpu
from jax.experimental.pallas import tpu_sc as plsc

N_NODES = 10000
N_PAD = 10240
E_PAD = 163840
DH = 512
NSLICE = 4
WSL = 128
EB = 128
ROWS_T = N_PAD // 16

_MESH = plsc.VectorSubcoreMesh(
    core_axis_name="c", subcore_axis_name="s", num_cores=2, num_subcores=16)


def _deg_body(col_hbm, ew_hbm, out_hbm, colv, ewv, zbuf, acc_sh):
    c = lax.axis_index("c")
    t = lax.axis_index("s")
    zero16 = jnp.zeros((16,), jnp.float32)
    for i in range(ROWS_T // 16):
        zbuf[pl.ds(i * 16, 16)] = zero16
    pltpu.sync_copy(zbuf, acc_sh.at[pl.ds(t * ROWS_T, ROWS_T)])
    plsc.subcore_barrier()
    npb = (E_PAD // 2) // 16 // EB

    def blk(b, carry):
        base = c * (E_PAD // 2) + t * (npb * EB) + b * EB
        pltpu.sync_copy(col_hbm.at[pl.ds(base, EB)], colv)
        pltpu.sync_copy(ew_hbm.at[pl.ds(base, EB)], ewv)
        pltpu.sync_copy(ewv, acc_sh.at[colv], add=True)
        return carry

    lax.fori_loop(0, npb, blk, 0)
    plsc.subcore_barrier()
    pltpu.sync_copy(acc_sh.at[pl.ds(t * ROWS_T, ROWS_T)],
                    out_hbm.at[pl.ds(c * N_PAD + t * ROWS_T, ROWS_T)])


_deg_call = pl.kernel(
    _deg_body,
    out_type=jax.ShapeDtypeStruct((2 * N_PAD,), jnp.float32),
    mesh=_MESH,
    scratch_types=[
        pltpu.VMEM((EB,), jnp.int32),
        pltpu.VMEM((EB,), jnp.float32),
        pltpu.VMEM((ROWS_T,), jnp.float32),
        pltpu.VMEM_SHARED((N_PAD,), jnp.float32),
    ],
)


_NBLK = (E_PAD // 16) // EB
_CH = 16


def _agg_body(g_hbm, rows4_hbm, col_hbm, ew_hbm, out_hbm,
              idxv, colv, ewv, buf0, buf1, acc_sh, gs0, gs1):
    c = lax.axis_index("c")
    t = lax.axis_index("s")

    def scale(b, buf):
        def grp(j, cc):
            w16 = ewv[b, pl.ds(j * 16, 16)]
            for l in range(16):
                w = w16[l]
                e = j * 16 + l
                for q in range(8):
                    sl = pl.ds(q * 16, 16)
                    buf[e, sl] = buf[e, sl] * w
            return cc
        lax.fori_loop(0, 8, grp, 0)

    for k in range(2):
        s = 2 * c + k
        srow = s * N_PAD
        pltpu.sync_copy(g_hbm.at[pl.ds(srow + t * ROWS_T, ROWS_T)],
                        acc_sh.at[pl.ds(t * ROWS_T, ROWS_T)])
        plsc.subcore_barrier()

        def chunk(ch, carry0):
            blk0 = t * _NBLK + ch * _CH
            pltpu.sync_copy(
                rows4_hbm.at[pl.ds(s * (E_PAD // EB) + blk0, _CH)], idxv)
            pltpu.sync_copy(col_hbm.at[pl.ds(blk0, _CH)], colv)
            pltpu.sync_copy(ew_hbm.at[pl.ds(blk0, _CH)], ewv)

            pltpu.async_copy(g_hbm.at[idxv.at[0]], buf0, gs0)

            def pair(m, carry):
                b0 = 2 * m
                b1 = 2 * m + 1
                pltpu.async_copy(g_hbm.at[idxv.at[b1]], buf1, gs1)
                pltpu.make_async_copy(g_hbm.at[idxv.at[b0]], buf0, gs0).wait()
                scale(b0, buf0)
                pltpu.sync_copy(buf0, acc_sh.at[colv.at[b0]], add=True)

                @pl.when(b1 + 1 < _CH)
                def _():
                    pltpu.async_copy(g_hbm.at[idxv.at[b1 + 1]], buf0, gs0)

                pltpu.make_async_copy(g_hbm.at[idxv.at[b1]], buf1, gs1).wait()
                scale(b1, buf1)
                pltpu.sync_copy(buf1, acc_sh.at[colv.at[b1]], add=True)
                return carry

            lax.fori_loop(0, _CH // 2, pair, 0)
            return carry0

        lax.fori_loop(0, _NBLK // _CH, chunk, 0)
        plsc.subcore_barrier()
        pltpu.sync_copy(acc_sh.at[pl.ds(t * ROWS_T, ROWS_T)],
                        out_hbm.at[pl.ds(srow + t * ROWS_T, ROWS_T)])
        plsc.subcore_barrier()


_agg_call = pl.kernel(
    _agg_body,
    out_type=jax.ShapeDtypeStruct((NSLICE * N_PAD, WSL), jnp.float32),
    mesh=_MESH,
    scratch_types=[
        pltpu.VMEM((_CH, EB), jnp.int32),
        pltpu.VMEM((_CH, EB), jnp.int32),
        pltpu.VMEM((_CH, EB), jnp.float32),
        pltpu.VMEM((EB, WSL), jnp.float32),
        pltpu.VMEM((EB, WSL), jnp.float32),
        pltpu.VMEM_SHARED((N_PAD, WSL), jnp.float32),
        pltpu.SemaphoreType.DMA,
        pltpu.SemaphoreType.DMA,
    ],
)


_BN = 1024


def _dinv_of(deg_ref):
    return lax.rsqrt(deg_ref[0, :] + deg_ref[1, :] + 1.0)


def _elu(h):
    return jnp.where(h > 0, h, jnp.exp(h) - 1.0)


def _mm1_body(deg_ref, x_ref, w_ref, out_ref):
    dinv = _dinv_of(deg_ref)
    h = jnp.dot(x_ref[...], w_ref[...], preferred_element_type=jnp.float32)
    for tt in range(NSLICE):
        out_ref[tt] = dinv[:, None] * h[:, tt * WSL:(tt + 1) * WSL]


def _mid_body(deg_ref, acc_ref, b_ref, w_ref, out_ref):
    dinv = _dinv_of(deg_ref)
    acts = [
        _elu(dinv[:, None] * acc_ref[si] + b_ref[0, si * WSL:(si + 1) * WSL][None, :])
        for si in range(NSLICE)
    ]
    act = jnp.concatenate(acts, axis=1)
    h = jnp.dot(act, w_ref[...], preferred_element_type=jnp.float32)
    for tt in range(NSLICE):
        out_ref[tt] = dinv[:, None] * h[:, tt * WSL:(tt + 1) * WSL]


def _fin_body(deg_ref, acc_ref, b_ref, w_ref, blin_ref, out_ref):
    dinv = _dinv_of(deg_ref)
    acts = [
        _elu(dinv[:, None] * acc_ref[si] + b_ref[0, si * WSL:(si + 1) * WSL][None, :])
        for si in range(NSLICE)
    ]
    act = jnp.concatenate(acts, axis=1)
    out_ref[...] = (
        jnp.dot(act, w_ref[...], preferred_element_type=jnp.float32)
        + blin_ref[0, :][None, :])


_deg_spec = pl.BlockSpec((2, _BN), lambda i: (0, i))
_acc_spec = pl.BlockSpec((NSLICE, _BN, WSL), lambda i: (0, i, 0))


def _mm1(deg2, x_p, W1):
    return pl.pallas_call(
        _mm1_body,
        grid=(N_PAD // _BN,),
        in_specs=[
            _deg_spec,
            pl.BlockSpec((_BN, 256), lambda i: (i, 0)),
            pl.BlockSpec((256, DH), lambda i: (0, 0)),
        ],
        out_specs=_acc_spec,
        out_shape=jax.ShapeDtypeStruct((NSLICE, N_PAD, WSL), jnp.float32),
    )(deg2, x_p, W1)


def _mid(deg2, acc, b_prev, W):
    return pl.pallas_call(
        _mid_body,
        grid=(N_PAD // _BN,),
        in_specs=[
            _deg_spec,
            _acc_spec,
            pl.BlockSpec((1, DH), lambda i: (0, 0)),
            pl.BlockSpec((DH, DH), lambda i: (0, 0)),
        ],
        out_specs=_acc_spec,
        out_shape=jax.ShapeDtypeStruct((NSLICE, N_PAD, WSL), jnp.float32),
    )(deg2, acc, b_prev, W)


def _fin(deg2, acc, b_prev, W_lin, b_lin):
    return pl.pallas_call(
        _fin_body,
        grid=(N_PAD // _BN,),
        in_specs=[
            _deg_spec,
            _acc_spec,
            pl.BlockSpec((1, DH), lambda i: (0, 0)),
            pl.BlockSpec((DH, 256), lambda i: (0, 0)),
            pl.BlockSpec((1, 256), lambda i: (0, 0)),
        ],
        out_specs=pl.BlockSpec((_BN, 256), lambda i: (i, 0)),
        out_shape=jax.ShapeDtypeStruct((N_PAD, 256), jnp.float32),
    )(deg2, acc, b_prev, W_lin, b_lin)


def kernel(x, edge_index, edge_feats, W1, b1, W_hidden, b_hidden, W_lin, b_lin):
    e = edge_index.shape[1]
    pe = E_PAD - e
    row_p = jnp.pad(edge_index[0], (0, pe))
    col_p = jnp.pad(edge_index[1], (0, pe))
    ew_p = jnp.pad(edge_feats, (0, pe))
    offs = (jnp.arange(NSLICE, dtype=jnp.int32) * N_PAD)[:, None]
    rows4 = (row_p[None, :] + offs).reshape(-1, EB)
    col2 = col_p.reshape(-1, EB)
    ew2 = ew_p.reshape(-1, EB)
    x_p = jnp.pad(x, ((0, N_PAD - N_NODES), (0, 0)))

    deg2 = _deg_call(col_p, ew_p).reshape(2, N_PAD)

    g = _mm1(deg2, x_p, W1)
    acc = _agg_call(g.reshape(NSLICE * N_PAD, WSL), rows4, col2, ew2)
    acc = acc.reshape(NSLICE, N_PAD, WSL)
    b_prev = b1
    for i in range(W_hidden.shape[0]):
        g = _mid(deg2, acc, b_prev.reshape(1, DH), W_hidden[i])
        acc = _agg_call(g.reshape(NSLICE * N_PAD, WSL), rows4, col2, ew2)
        acc = acc.reshape(NSLICE, N_PAD, WSL)
        b_prev = b_hidden[i]
    out = _fin(deg2, acc, b_prev.reshape(1, DH), W_lin, b_lin.reshape(1, -1))
    return out[:N_NODES]
